# baseline (speedup 1.0000x reference)
import functools

import jax
import jax.numpy as jnp
from jax import lax
from jax.experimental import pallas as pl
from jax.experimental.pallas import tpu as pltpu
from jax.experimental.pallas import tpu_sc as plsc

NUM_EXPERTS = 8
TOP_K = 2
D_MODEL = 1024
D_FF = 4096
TOKENS = 2048
CAP = (TOP_K * TOKENS) // NUM_EXPERTS
SLOTS = NUM_EXPERTS * CAP
SLOTS_PAD = SLOTS + 8

NC = 2
NS = 16
NW = NC * NS
TOK_PER_W = TOKENS // NW

F_BLK = 1024
N_FBLK = D_FF // F_BLK


def _router_body(x_ref, wr_ref, br_ref, idx_ref, w_ref, wexp_ref):
    x = x_ref[...]
    wr = wr_ref[...]
    logits = jnp.dot(x, wr, preferred_element_type=jnp.float32)
    logits = logits + br_ref[...]
    lane = lax.broadcasted_iota(jnp.int32, logits.shape, 1)
    neg = jnp.where(lane < NUM_EXPERTS, logits, -1e30)

    m1 = jnp.max(neg, axis=1, keepdims=True)
    a1 = jnp.min(jnp.where(neg == m1, lane, 127), axis=1, keepdims=True)
    neg2 = jnp.where(lane == a1, -1e30, neg)
    m2 = jnp.max(neg2, axis=1, keepdims=True)
    a2 = jnp.min(jnp.where(neg2 == m2, lane, 127), axis=1, keepdims=True)

    w1 = 1.0 / (1.0 + jnp.exp(m2 - m1))
    w2 = 1.0 - w1

    e8 = lax.broadcasted_iota(jnp.int32, (TOKENS, NUM_EXPERTS), 1)
    ohs = ((a1 == e8).astype(jnp.float32) + (a2 == e8).astype(jnp.float32))
    r2 = lax.broadcasted_iota(jnp.int32, (TOKENS, TOKENS), 0)
    c2 = lax.broadcasted_iota(jnp.int32, (TOKENS, TOKENS), 1)
    tri = (r2 >= c2).astype(jnp.float32)
    S = jnp.dot(tri, ohs, preferred_element_type=jnp.float32)
    pos1 = jnp.sum(jnp.where(e8 == a1, S, 0.0), axis=1, keepdims=True) - 1.0
    pos2 = jnp.sum(jnp.where(e8 == a2, S, 0.0), axis=1, keepdims=True) - 1.0
    pos1 = pos1.astype(jnp.int32)
    pos2 = pos2.astype(jnp.int32)

    keep1 = pos1 < CAP
    keep2 = pos2 < CAP
    slot1 = a1 * CAP + pos1
    slot2 = a2 * CAP + pos2
    d1s = jnp.where(keep1, slot1, SLOTS)
    d2s = jnp.where(keep2, slot2, SLOTS)
    row = lax.broadcasted_iota(jnp.int32, (TOKENS, 1), 0)
    e_first = jnp.sum(jnp.where(row == 0, a1, 0)) * CAP
    d1c = jnp.where(keep1, slot1, e_first)
    d2c = jnp.where(keep2, slot2, e_first)
    w1e = jnp.where(keep1, w1, 0.0)
    w2e = jnp.where(keep2, w2, 0.0)

    l8 = lax.broadcasted_iota(jnp.int32, (TOKENS, 8), 1)
    idx_ref[...] = (jnp.where(l8 == 0, d1s, 0) + jnp.where(l8 == 1, d2s, 0)
                    + jnp.where(l8 == 2, d1c, 0) + jnp.where(l8 == 3, d2c, 0))
    w_ref[...] = jnp.where(l8 == 0, w1e, 0.0) + jnp.where(l8 == 1, w2e, 0.0)
    l32 = lax.broadcasted_iota(jnp.int32, (TOKENS, 32), 1)
    wexp_ref[...] = jnp.where(l32 < 16, w1e, w2e)


def _router(x, wr_pad, br_pad, interpret=False):
    return pl.pallas_call(
        _router_body,
        out_shape=(
            jax.ShapeDtypeStruct((TOKENS, 8), jnp.int32),
            jax.ShapeDtypeStruct((TOKENS, 8), jnp.float32),
            jax.ShapeDtypeStruct((TOKENS, 32), jnp.float32),
        ),
        interpret=interpret,
    )(x, wr_pad, br_pad)


def _dispatch_body(x_hbm, d1_hbm, d2_hbm, out_hbm, xbuf, i1, i2, sem1, sem2):
    wid = lax.axis_index("s") * NC + lax.axis_index("c")
    base = wid * TOK_PER_W
    pltpu.sync_copy(x_hbm.at[pl.ds(base, TOK_PER_W)], xbuf)
    pltpu.sync_copy(d1_hbm.at[pl.ds(base, TOK_PER_W)], i1)
    pltpu.sync_copy(d2_hbm.at[pl.ds(base, TOK_PER_W)], i2)
    c1 = pltpu.async_copy(xbuf, out_hbm.at[i1], sem1)
    c2 = pltpu.async_copy(xbuf, out_hbm.at[i2], sem2)
    c1.wait()
    c2.wait()


def _dispatch(x, d1s, d2s):
    mesh = plsc.VectorSubcoreMesh(core_axis_name="c", subcore_axis_name="s")
    fn = pl.kernel(
        _dispatch_body,
        mesh=mesh,
        out_type=jax.ShapeDtypeStruct((SLOTS_PAD, D_MODEL), jnp.float32),
        scratch_types=[
            pltpu.VMEM((TOK_PER_W, D_MODEL), jnp.float32),
            pltpu.VMEM((TOK_PER_W,), jnp.int32),
            pltpu.VMEM((TOK_PER_W,), jnp.int32),
            pltpu.SemaphoreType.DMA,
            pltpu.SemaphoreType.DMA,
        ],
    )
    return fn(x, d1s, d2s)


def _ffn_body(ein_ref, wg_ref, wu_ref, wd_ref, bg_ref, bu_ref, bd_ref, out_ref):
    e = pl.program_id(0)
    f = pl.program_id(1)
    xin = ein_ref[...].astype(jnp.bfloat16)
    wg = wg_ref[0].astype(jnp.bfloat16)
    wu = wu_ref[0].astype(jnp.bfloat16)
    wd = wd_ref[0].astype(jnp.bfloat16)
    g = jnp.dot(xin, wg, preferred_element_type=jnp.float32)
    u = jnp.dot(xin, wu, preferred_element_type=jnp.float32)
    bg = bg_ref[e, pl.ds(f * F_BLK, F_BLK)][None, :]
    bu = bu_ref[e, pl.ds(f * F_BLK, F_BLK)][None, :]
    g = g + bg
    u = u + bu
    h = g * jax.nn.sigmoid(g) * u
    contrib = jnp.dot(h.astype(jnp.bfloat16), wd, preferred_element_type=jnp.float32)

    @pl.when(f == 0)
    def _():
        out_ref[...] = contrib + bd_ref[e][None, :]

    @pl.when(f > 0)
    def _():
        out_ref[...] += contrib


def _ffn(ein, Wg, bg, Wu, bu, Wd, bd, interpret=False):
    return pl.pallas_call(
        _ffn_body,
        grid=(NUM_EXPERTS, N_FBLK),
        in_specs=[
            pl.BlockSpec((CAP, D_MODEL), lambda e, f: (e, 0)),
            pl.BlockSpec((1, D_MODEL, F_BLK), lambda e, f: (e, 0, f)),
            pl.BlockSpec((1, D_MODEL, F_BLK), lambda e, f: (e, 0, f)),
            pl.BlockSpec((1, F_BLK, D_MODEL), lambda e, f: (e, f, 0)),
            pl.BlockSpec((NUM_EXPERTS, D_FF), lambda e, f: (0, 0)),
            pl.BlockSpec((NUM_EXPERTS, D_FF), lambda e, f: (0, 0)),
            pl.BlockSpec((NUM_EXPERTS, D_MODEL), lambda e, f: (0, 0)),
        ],
        out_specs=pl.BlockSpec((CAP, D_MODEL), lambda e, f: (e, 0)),
        out_shape=jax.ShapeDtypeStruct((SLOTS, D_MODEL), jnp.float32),
        interpret=interpret,
    )(ein, Wg, Wu, Wd, bg, bu, bd)


CHUNK = 32


def _combine_body(eo_hbm, d1_hbm, d2_hbm, wexp_hbm, out_hbm,
                  abuf, bbuf, wbuf, i1, i2, sem1, sem2):
    wid = lax.axis_index("s") * NC + lax.axis_index("c")
    for ch in range(TOK_PER_W // CHUNK):
        base = wid * TOK_PER_W + ch * CHUNK
        pltpu.sync_copy(d1_hbm.at[pl.ds(base, CHUNK)], i1)
        pltpu.sync_copy(d2_hbm.at[pl.ds(base, CHUNK)], i2)
        pltpu.sync_copy(wexp_hbm.at[pl.ds(base, CHUNK)], wbuf)
        c1 = pltpu.async_copy(eo_hbm.at[i1], abuf, sem1)
        c2 = pltpu.async_copy(eo_hbm.at[i2], bbuf, sem2)
        c1.wait()
        c2.wait()

        def row_body(r, _):
            wa = wbuf[r, pl.ds(0, 16)]
            wb = wbuf[r, pl.ds(16, 16)]
            for j in range(D_MODEL // 16):
                cs = j * 16
                a = abuf[r, pl.ds(cs, 16)]
                b = bbuf[r, pl.ds(cs, 16)]
                abuf[r, pl.ds(cs, 16)] = a * wa + b * wb
            return 0

        lax.fori_loop(0, CHUNK, row_body, 0)
        pltpu.sync_copy(abuf, out_hbm.at[pl.ds(base, CHUNK)])


def _combine(eo, d1c, d2c, wexp):
    mesh = plsc.VectorSubcoreMesh(core_axis_name="c", subcore_axis_name="s")
    fn = pl.kernel(
        _combine_body,
        mesh=mesh,
        out_type=jax.ShapeDtypeStruct((TOKENS, D_MODEL), jnp.float32),
        scratch_types=[
            pltpu.VMEM((CHUNK, D_MODEL), jnp.float32),
            pltpu.VMEM((CHUNK, D_MODEL), jnp.float32),
            pltpu.VMEM((CHUNK, 32), jnp.float32),
            pltpu.VMEM((CHUNK,), jnp.int32),
            pltpu.VMEM((CHUNK,), jnp.int32),
            pltpu.SemaphoreType.DMA,
            pltpu.SemaphoreType.DMA,
        ],
    )
    return fn(eo, d1c, d2c, wexp)


def kernel(x, Wr, br, Wg, bg, Wu, bu, Wd, bd):
    wr_pad = jnp.pad(Wr, ((0, 0), (0, 128 - NUM_EXPERTS)))
    br_pad = jnp.pad(br, (0, 128 - NUM_EXPERTS))[None, :]
    idx, w, wexp = _router(x, wr_pad, br_pad)
    d1s = idx[:, 0]
    d2s = idx[:, 1]
    d1c = idx[:, 2]
    d2c = idx[:, 3]
    ein = _dispatch(x, d1s, d2s)
    eo = _ffn(ein, Wg, bg, Wu, bu, Wd, bd)
    return _combine(eo, d1c, d2c, wexp)


# speedup vs baseline: 1.0016x; 1.0016x over previous
import jax
import jax.numpy as jnp
from jax import lax
from jax.experimental import pallas as pl
from jax.experimental.pallas import tpu as pltpu
from jax.experimental.pallas import tpu_sc as plsc

NUM_EXPERTS = 8
TOP_K = 2
D_MODEL = 1024
D_FF = 4096
TOKENS = 2048
CAP = (TOP_K * TOKENS) // NUM_EXPERTS
SLOTS = NUM_EXPERTS * CAP
SLOTS_PAD = SLOTS + 8

NC = 2
NS = 16
NW = NC * NS
TOK_PER_W = TOKENS // NW

F_BLK = 1024
N_FBLK = D_FF // F_BLK


def _router_body(x_ref, wr_ref, br_ref, idx_ref, w_ref, wexp_ref):
    x = x_ref[...]
    wr = wr_ref[...]
    logits = jnp.dot(x, wr, preferred_element_type=jnp.float32)
    logits = logits + br_ref[...]
    lane = lax.broadcasted_iota(jnp.int32, logits.shape, 1)
    neg = jnp.where(lane < NUM_EXPERTS, logits, -1e30)

    m1 = jnp.max(neg, axis=1, keepdims=True)
    a1 = jnp.min(jnp.where(neg == m1, lane, 127), axis=1, keepdims=True)
    neg2 = jnp.where(lane == a1, -1e30, neg)
    m2 = jnp.max(neg2, axis=1, keepdims=True)
    a2 = jnp.min(jnp.where(neg2 == m2, lane, 127), axis=1, keepdims=True)

    w1 = 1.0 / (1.0 + jnp.exp(m2 - m1))
    w2 = 1.0 - w1

    e8 = lax.broadcasted_iota(jnp.int32, (TOKENS, NUM_EXPERTS), 1)
    ohs = ((a1 == e8).astype(jnp.float32) + (a2 == e8).astype(jnp.float32))
    r2 = lax.broadcasted_iota(jnp.int32, (TOKENS, TOKENS), 0)
    c2 = lax.broadcasted_iota(jnp.int32, (TOKENS, TOKENS), 1)
    tri = (r2 >= c2).astype(jnp.float32)
    S = jnp.dot(tri, ohs, preferred_element_type=jnp.float32)
    pos1 = jnp.sum(jnp.where(e8 == a1, S, 0.0), axis=1, keepdims=True) - 1.0
    pos2 = jnp.sum(jnp.where(e8 == a2, S, 0.0), axis=1, keepdims=True) - 1.0
    pos1 = pos1.astype(jnp.int32)
    pos2 = pos2.astype(jnp.int32)

    keep1 = pos1 < CAP
    keep2 = pos2 < CAP
    slot1 = a1 * CAP + pos1
    slot2 = a2 * CAP + pos2
    d1s = jnp.where(keep1, slot1, SLOTS)
    d2s = jnp.where(keep2, slot2, SLOTS)
    row = lax.broadcasted_iota(jnp.int32, (TOKENS, 1), 0)
    e_first = jnp.sum(jnp.where(row == 0, a1, 0)) * CAP
    d1c = jnp.where(keep1, slot1, e_first)
    d2c = jnp.where(keep2, slot2, e_first)
    w1e = jnp.where(keep1, w1, 0.0)
    w2e = jnp.where(keep2, w2, 0.0)

    l8 = lax.broadcasted_iota(jnp.int32, (TOKENS, 8), 1)
    idx_ref[...] = (jnp.where(l8 == 0, d1s, 0) + jnp.where(l8 == 1, d2s, 0)
                    + jnp.where(l8 == 2, d1c, 0) + jnp.where(l8 == 3, d2c, 0))
    w_ref[...] = jnp.where(l8 == 0, w1e, 0.0) + jnp.where(l8 == 1, w2e, 0.0)
    l32 = lax.broadcasted_iota(jnp.int32, (TOKENS, 32), 1)
    wexp_ref[...] = jnp.where(l32 < 16, w1e, w2e)


def _router(x, wr_pad, br_pad, interpret=False):
    return pl.pallas_call(
        _router_body,
        out_shape=(
            jax.ShapeDtypeStruct((TOKENS, 8), jnp.int32),
            jax.ShapeDtypeStruct((TOKENS, 8), jnp.float32),
            jax.ShapeDtypeStruct((TOKENS, 32), jnp.float32),
        ),
        interpret=interpret,
    )(x, wr_pad, br_pad)


def _dispatch_body(x_hbm, d1_hbm, d2_hbm, out_hbm, xbuf, i1, i2, sem1, sem2):
    wid = lax.axis_index("s") * NC + lax.axis_index("c")
    base = wid * TOK_PER_W
    pltpu.sync_copy(x_hbm.at[pl.ds(base, TOK_PER_W)], xbuf)
    pltpu.sync_copy(d1_hbm.at[pl.ds(base, TOK_PER_W)], i1)
    pltpu.sync_copy(d2_hbm.at[pl.ds(base, TOK_PER_W)], i2)
    c1 = pltpu.async_copy(xbuf, out_hbm.at[i1], sem1)
    c2 = pltpu.async_copy(xbuf, out_hbm.at[i2], sem2)
    c1.wait()
    c2.wait()


def _dispatch(x, d1s, d2s):
    mesh = plsc.VectorSubcoreMesh(core_axis_name="c", subcore_axis_name="s")
    fn = pl.kernel(
        _dispatch_body,
        mesh=mesh,
        out_type=jax.ShapeDtypeStruct((SLOTS_PAD, D_MODEL), jnp.float32),
        scratch_types=[
            pltpu.VMEM((TOK_PER_W, D_MODEL), jnp.float32),
            pltpu.VMEM((TOK_PER_W,), jnp.int32),
            pltpu.VMEM((TOK_PER_W,), jnp.int32),
            pltpu.SemaphoreType.DMA,
            pltpu.SemaphoreType.DMA,
        ],
    )
    return fn(x, d1s, d2s)


def _ffn_body(ein_ref, wg_ref, wu_ref, wd_ref, bg_ref, bu_ref, bd_ref, out_ref):
    e = pl.program_id(0)
    f = pl.program_id(1)
    xin = ein_ref[...].astype(jnp.bfloat16)
    wg = wg_ref[0].astype(jnp.bfloat16)
    wu = wu_ref[0].astype(jnp.bfloat16)
    wd = wd_ref[0].astype(jnp.bfloat16)
    g = jnp.dot(xin, wg, preferred_element_type=jnp.float32)
    u = jnp.dot(xin, wu, preferred_element_type=jnp.float32)
    bg = bg_ref[e, pl.ds(f * F_BLK, F_BLK)][None, :]
    bu = bu_ref[e, pl.ds(f * F_BLK, F_BLK)][None, :]
    g = g + bg
    u = u + bu
    h = g * jax.nn.sigmoid(g) * u
    contrib = jnp.dot(h.astype(jnp.bfloat16), wd, preferred_element_type=jnp.float32)

    @pl.when(f == 0)
    def _():
        out_ref[...] = contrib + bd_ref[e][None, :]

    @pl.when(f > 0)
    def _():
        out_ref[...] += contrib


def _ffn(ein, Wg, bg, Wu, bu, Wd, bd, interpret=False):
    return pl.pallas_call(
        _ffn_body,
        grid=(NUM_EXPERTS, N_FBLK),
        in_specs=[
            pl.BlockSpec((CAP, D_MODEL), lambda e, f: (e, 0)),
            pl.BlockSpec((1, D_MODEL, F_BLK), lambda e, f: (e, 0, f)),
            pl.BlockSpec((1, D_MODEL, F_BLK), lambda e, f: (e, 0, f)),
            pl.BlockSpec((1, F_BLK, D_MODEL), lambda e, f: (e, f, 0)),
            pl.BlockSpec((NUM_EXPERTS, D_FF), lambda e, f: (0, 0)),
            pl.BlockSpec((NUM_EXPERTS, D_FF), lambda e, f: (0, 0)),
            pl.BlockSpec((NUM_EXPERTS, D_MODEL), lambda e, f: (0, 0)),
        ],
        out_specs=pl.BlockSpec((CAP, D_MODEL), lambda e, f: (e, 0)),
        out_shape=jax.ShapeDtypeStruct((SLOTS, D_MODEL), jnp.float32),
        interpret=interpret,
    )(ein, Wg, Wu, Wd, bg, bu, bd)


CHUNK = 32


def _combine_body(eo_hbm, d1_hbm, d2_hbm, wexp_hbm, out_hbm,
                  abuf, bbuf, wbuf, i1, i2, sem1, sem2):
    wid = lax.axis_index("s") * NC + lax.axis_index("c")
    for ch in range(TOK_PER_W // CHUNK):
        base = wid * TOK_PER_W + ch * CHUNK
        pltpu.sync_copy(d1_hbm.at[pl.ds(base, CHUNK)], i1)
        pltpu.sync_copy(d2_hbm.at[pl.ds(base, CHUNK)], i2)
        pltpu.sync_copy(wexp_hbm.at[pl.ds(base, CHUNK)], wbuf)
        c1 = pltpu.async_copy(eo_hbm.at[i1], abuf, sem1)
        c2 = pltpu.async_copy(eo_hbm.at[i2], bbuf, sem2)
        c1.wait()
        c2.wait()

        def row_body(r, _):
            wa = wbuf[r, pl.ds(0, 16)]
            wb = wbuf[r, pl.ds(16, 16)]
            for j in range(D_MODEL // 16):
                cs = j * 16
                a = abuf[r, pl.ds(cs, 16)]
                b = bbuf[r, pl.ds(cs, 16)]
                abuf[r, pl.ds(cs, 16)] = a * wa + b * wb
            return 0

        lax.fori_loop(0, CHUNK, row_body, 0)
        pltpu.sync_copy(abuf, out_hbm.at[pl.ds(base, CHUNK)])


def _combine(eo, d1c, d2c, wexp):
    mesh = plsc.VectorSubcoreMesh(core_axis_name="c", subcore_axis_name="s")
    fn = pl.kernel(
        _combine_body,
        mesh=mesh,
        out_type=jax.ShapeDtypeStruct((TOKENS, D_MODEL), jnp.float32),
        scratch_types=[
            pltpu.VMEM((CHUNK, D_MODEL), jnp.float32),
            pltpu.VMEM((CHUNK, D_MODEL), jnp.float32),
            pltpu.VMEM((CHUNK, 32), jnp.float32),
            pltpu.VMEM((CHUNK,), jnp.int32),
            pltpu.VMEM((CHUNK,), jnp.int32),
            pltpu.SemaphoreType.DMA,
            pltpu.SemaphoreType.DMA,
        ],
    )
    return fn(eo, d1c, d2c, wexp)


def kernel(x, Wr, br, Wg, bg, Wu, bu, Wd, bd):
    wr_pad = jnp.pad(Wr, ((0, 0), (0, 128 - NUM_EXPERTS)))
    br_pad = jnp.pad(br, (0, 128 - NUM_EXPERTS))[None, :]
    idx, w, wexp = _router(x, wr_pad, br_pad)
    d1s = idx[:, 0]
    d2s = idx[:, 1]
    d1c = idx[:, 2]
    d2c = idx[:, 3]
    ein = _dispatch(x, d1s, d2s)
    eo = _ffn(ein, Wg, bg, Wu, bu, Wd, bd)
    return _combine(eo, d1c, d2c, wexp)
